# baseline (speedup 1.0000x reference)
import functools

import jax
import jax.numpy as jnp
from jax import lax
from jax.experimental import pallas as pl
from jax.experimental.pallas import tpu as pltpu
from jax.experimental.pallas import tpu_sc as plsc

_LANES = 16
_NUM_WORKERS = 32
_ROWS_PER_CHUNK = 8
_NBUF = 2


@jax.jit
def _sc_lookup(vocab_map, tokens):
    n_rows, seq = tokens.shape
    rows_per_worker = n_rows // _NUM_WORKERS
    n_chunks = rows_per_worker // _ROWS_PER_CHUNK
    half_chunks = n_chunks // 2
    half_cols = rows_per_worker // 2
    mesh = plsc.VectorSubcoreMesh(
        core_axis_name="c", subcore_axis_name="s", num_cores=2, num_subcores=16
    )

    @functools.partial(
        pl.kernel,
        out_type=jax.ShapeDtypeStruct((seq, n_rows), jnp.float32),
        mesh=mesh,
        scratch_types=[
            pltpu.VMEM(vocab_map.shape, jnp.float32),
            [pltpu.VMEM((_ROWS_PER_CHUNK, seq), jnp.int32) for _ in range(_NBUF)],
            pltpu.VMEM((seq, rows_per_worker), jnp.float32),
            pltpu.SemaphoreType.DMA,
            [pltpu.SemaphoreType.DMA for _ in range(_NBUF)],
            [pltpu.SemaphoreType.DMA for _ in range(2)],
        ],
        compiler_params=pltpu.CompilerParams(
            use_tc_tiling_on_sc=False, needs_layout_passes=False
        ),
    )
    def body(table_hbm, tok_hbm, out_hbm, table_v, idx_v, out_v,
             sem_tab, sem_in, sem_out):
        wid = lax.axis_index("s") * 2 + lax.axis_index("c")
        base = wid * rows_per_worker

        cp_tab = pltpu.async_copy(table_hbm, table_v, sem_tab)
        in_cps = [None] * _NBUF
        for c in range(min(_NBUF, n_chunks)):
            in_cps[c] = pltpu.async_copy(
                tok_hbm.at[pl.ds(base + c * _ROWS_PER_CHUNK, _ROWS_PER_CHUNK), :],
                idx_v[c], sem_in[c],
            )
        cp_tab.wait()

        lane_iota = lax.iota(jnp.int32, _LANES)
        cols = [k * _LANES for k in range(seq // _LANES)]
        if seq % _LANES:
            cols.append(seq - _LANES)

        out_cps = [None, None]
        for c in range(n_chunks):
            b = c % _NBUF
            in_cps[b].wait()
            col0 = c * _ROWS_PER_CHUNK

            @plsc.parallel_loop(0, _ROWS_PER_CHUNK, step=1, unroll=2)
            def _(r):
                r_vec = jnp.full((_LANES,), col0, jnp.int32) + r
                for col in cols:
                    sl = pl.ds(col, _LANES)
                    vals = plsc.load_gather(table_v, [idx_v[b][r, sl]])
                    plsc.store_scatter(out_v, [lane_iota + col, r_vec], vals)

            nxt = c + _NBUF
            if nxt < n_chunks:
                in_cps[b] = pltpu.async_copy(
                    tok_hbm.at[pl.ds(base + nxt * _ROWS_PER_CHUNK, _ROWS_PER_CHUNK), :],
                    idx_v[b], sem_in[b],
                )
            if c == half_chunks - 1:
                out_cps[0] = pltpu.async_copy(
                    out_v.at[:, pl.ds(0, half_cols)],
                    out_hbm.at[:, pl.ds(base, half_cols)],
                    sem_out[0],
                )
        out_cps[1] = pltpu.async_copy(
            out_v.at[:, pl.ds(half_cols, half_cols)],
            out_hbm.at[:, pl.ds(base + half_cols, half_cols)],
            sem_out[1],
        )
        out_cps[0].wait()
        out_cps[1].wait()

    return body(vocab_map, tokens).T


def kernel(tokens, start_idxs, end_idxs, vocab_map):
    return _sc_lookup(vocab_map, tokens), start_idxs, end_idxs


# speedup vs baseline: 1.1566x; 1.1566x over previous
import functools

import jax
import jax.numpy as jnp
from jax import lax
from jax.experimental import pallas as pl
from jax.experimental.pallas import tpu as pltpu
from jax.experimental.pallas import tpu_sc as plsc

_LANES = 16
_NUM_WORKERS = 32
_CHUNK = 6400
_NBUF = 2


@functools.partial(jax.jit, static_argnums=(2,))
def _sc_lookup(vocab_map, flat_tokens, n_per_worker):
    n_chunks = n_per_worker // _CHUNK
    mesh = plsc.VectorSubcoreMesh(
        core_axis_name="c", subcore_axis_name="s", num_cores=2, num_subcores=16
    )

    @functools.partial(
        pl.kernel,
        out_type=jax.ShapeDtypeStruct(flat_tokens.shape, jnp.float32),
        mesh=mesh,
        scratch_types=[
            pltpu.VMEM(vocab_map.shape, jnp.float32),
            [pltpu.VMEM((_CHUNK,), jnp.int32) for _ in range(_NBUF)],
            [pltpu.VMEM((_CHUNK,), jnp.float32) for _ in range(_NBUF)],
            pltpu.SemaphoreType.DMA,
            [pltpu.SemaphoreType.DMA for _ in range(_NBUF)],
            [pltpu.SemaphoreType.DMA for _ in range(_NBUF)],
        ],
        compiler_params=pltpu.CompilerParams(
            use_tc_tiling_on_sc=False, needs_layout_passes=False
        ),
    )
    def body(table_hbm, tok_hbm, out_hbm, table_v, idx_v, out_v,
             sem_tab, sem_in, sem_out):
        wid = lax.axis_index("s") * 2 + lax.axis_index("c")
        base = wid * n_per_worker

        cp_tab = pltpu.async_copy(table_hbm, table_v, sem_tab)
        in_cps = [None] * _NBUF
        out_cps = [None] * _NBUF
        for c in range(min(_NBUF, n_chunks)):
            in_cps[c] = pltpu.async_copy(
                tok_hbm.at[pl.ds(base + c * _CHUNK, _CHUNK)],
                idx_v[c], sem_in[c],
            )
        cp_tab.wait()

        for c in range(n_chunks):
            b = c % _NBUF
            in_cps[b].wait()
            if out_cps[b] is not None:
                out_cps[b].wait()

            @plsc.parallel_loop(0, _CHUNK, step=_LANES, unroll=8)
            def _(i):
                sl = pl.ds(i, _LANES)
                out_v[b][sl] = plsc.load_gather(table_v, [idx_v[b][sl]])

            out_cps[b] = pltpu.async_copy(
                out_v[b], out_hbm.at[pl.ds(base + c * _CHUNK, _CHUNK)],
                sem_out[b],
            )
            nxt = c + _NBUF
            if nxt < n_chunks:
                in_cps[b] = pltpu.async_copy(
                    tok_hbm.at[pl.ds(base + nxt * _CHUNK, _CHUNK)],
                    idx_v[b], sem_in[b],
                )
        for b in range(min(_NBUF, n_chunks)):
            if out_cps[b] is not None:
                out_cps[b].wait()

    return body(vocab_map, flat_tokens)


def kernel(tokens, start_idxs, end_idxs, vocab_map):
    b, s = tokens.shape
    n = b * s
    token_ids = _sc_lookup(vocab_map, tokens.reshape(n), n // _NUM_WORKERS)
    return token_ids.reshape(b, s), start_idxs, end_idxs


# speedup vs baseline: 1.3111x; 1.1336x over previous
import functools

import jax
import jax.numpy as jnp
from jax import lax
from jax.experimental import pallas as pl
from jax.experimental.pallas import tpu as pltpu
from jax.experimental.pallas import tpu_sc as plsc

_NUM_WORKERS = 32


@functools.partial(jax.jit, static_argnums=(2,))
def _sc_lookup(vocab_map, flat_tokens, n_per_worker):
    mesh = plsc.VectorSubcoreMesh(
        core_axis_name="c", subcore_axis_name="s", num_cores=2, num_subcores=16
    )

    @functools.partial(
        pl.kernel,
        out_type=jax.ShapeDtypeStruct(flat_tokens.shape, jnp.float32),
        mesh=mesh,
        scratch_types=[
            pltpu.VMEM_SHARED(vocab_map.shape, jnp.float32),
            pltpu.VMEM((n_per_worker,), jnp.int32),
            pltpu.VMEM((n_per_worker,), jnp.float32),
            pltpu.SemaphoreType.DMA,
            pltpu.SemaphoreType.DMA,
            pltpu.SemaphoreType.DMA,
        ],
        compiler_params=pltpu.CompilerParams(
            use_tc_tiling_on_sc=False, needs_layout_passes=False
        ),
    )
    def body(table_hbm, tok_hbm, out_hbm, table_sh, idx_v, out_v,
             sem_tab, sem_in, sem_out):
        sid = lax.axis_index("s")
        wid = sid * 2 + lax.axis_index("c")
        base = wid * n_per_worker

        cp_in = pltpu.async_copy(
            tok_hbm.at[pl.ds(base, n_per_worker)], idx_v, sem_in
        )

        @pl.when(sid == 0)
        def _():
            pltpu.async_copy(table_hbm, table_sh, sem_tab).wait()

        plsc.subcore_barrier()
        cp_in.wait()
        pltpu.async_copy(table_sh.at[idx_v], out_v, sem_out).wait()
        pltpu.async_copy(
            out_v, out_hbm.at[pl.ds(base, n_per_worker)], sem_in
        ).wait()

    return body(vocab_map, flat_tokens)


def kernel(tokens, start_idxs, end_idxs, vocab_map):
    b, s = tokens.shape
    n = b * s
    token_ids = _sc_lookup(vocab_map, tokens.reshape(n), n // _NUM_WORKERS)
    return token_ids.reshape(b, s), start_idxs, end_idxs


# speedup vs baseline: 1.3276x; 1.0126x over previous
import functools

import jax
import jax.numpy as jnp
from jax import lax
from jax.experimental import pallas as pl
from jax.experimental.pallas import tpu as pltpu
from jax.experimental.pallas import tpu_sc as plsc

_NUM_WORKERS = 32
_CHUNK = 6400
_NBUF = 2


@functools.partial(jax.jit, static_argnums=(2,))
def _sc_lookup(vocab_map, flat_tokens, n_per_worker):
    n_chunks = n_per_worker // _CHUNK
    mesh = plsc.VectorSubcoreMesh(
        core_axis_name="c", subcore_axis_name="s", num_cores=2, num_subcores=16
    )

    @functools.partial(
        pl.kernel,
        out_type=jax.ShapeDtypeStruct(flat_tokens.shape, jnp.float32),
        mesh=mesh,
        scratch_types=[
            pltpu.VMEM_SHARED(vocab_map.shape, jnp.float32),
            [pltpu.VMEM((_CHUNK,), jnp.int32) for _ in range(_NBUF)],
            [pltpu.VMEM((_CHUNK,), jnp.float32) for _ in range(_NBUF)],
            pltpu.SemaphoreType.DMA,
            [pltpu.SemaphoreType.DMA for _ in range(_NBUF)],
            pltpu.SemaphoreType.DMA,
            [pltpu.SemaphoreType.DMA for _ in range(_NBUF)],
        ],
        compiler_params=pltpu.CompilerParams(
            use_tc_tiling_on_sc=False, needs_layout_passes=False
        ),
    )
    def body(table_hbm, tok_hbm, out_hbm, table_sh, idx_v, out_v,
             sem_tab, sem_in, sem_g, sem_out):
        sid = lax.axis_index("s")
        wid = sid * 2 + lax.axis_index("c")
        base = wid * n_per_worker

        in_cps = [None] * _NBUF
        out_cps = [None] * _NBUF
        for c in range(min(_NBUF, n_chunks)):
            in_cps[c] = pltpu.async_copy(
                tok_hbm.at[pl.ds(base + c * _CHUNK, _CHUNK)],
                idx_v[c], sem_in[c],
            )

        @pl.when(sid == 0)
        def _():
            pltpu.async_copy(table_hbm, table_sh, sem_tab).wait()

        plsc.subcore_barrier()

        for c in range(n_chunks):
            b = c % _NBUF
            in_cps[b].wait()
            if out_cps[b] is not None:
                out_cps[b].wait()
            pltpu.async_copy(table_sh.at[idx_v[b]], out_v[b], sem_g).wait()
            out_cps[b] = pltpu.async_copy(
                out_v[b], out_hbm.at[pl.ds(base + c * _CHUNK, _CHUNK)],
                sem_out[b],
            )
            nxt = c + _NBUF
            if nxt < n_chunks:
                in_cps[b] = pltpu.async_copy(
                    tok_hbm.at[pl.ds(base + nxt * _CHUNK, _CHUNK)],
                    idx_v[b], sem_in[b],
                )
        for b in range(min(_NBUF, n_chunks)):
            if out_cps[b] is not None:
                out_cps[b].wait()

    return body(vocab_map, flat_tokens)


def kernel(tokens, start_idxs, end_idxs, vocab_map):
    b, s = tokens.shape
    n = b * s
    token_ids = _sc_lookup(vocab_map, tokens.reshape(n), n // _NUM_WORKERS)
    return token_ids.reshape(b, s), start_idxs, end_idxs
